# baseline (speedup 1.0000x reference)
import functools

import jax
import jax.numpy as jnp
from jax import lax
from jax.experimental import pallas as pl
from jax.experimental.pallas import tpu as pltpu
from jax.experimental.pallas import tpu_sc as plsc

_N = 10000
_E = 160000
_D = 256
_H = 128
_NC = 2
_NS = 16
_CE = 64
_NCHUNK = 157
_EPT = _NCHUNK * _CE
_EPAD = _NS * _EPT
_AROWS = _NCHUNK * _CE
_WB = 16
_NWB = _N // _WB
_WB_ITERS = (_NWB + _NS - 1) // _NS
_ZB_ITERS = (_NCHUNK + _NS - 1) // _NS
_F16 = _H // 16


def _proj_body(h_ref, wk_ref, wqv_ref, ws_ref, ok_ref, oqv_ref, os_ref):
    h = jnp.concatenate([h_ref[0], h_ref[1]], axis=1)
    for c in range(_NC):
        ok_ref[c] = jnp.dot(h, wk_ref[c], preferred_element_type=jnp.float32)
        oqv_ref[c] = jnp.dot(h, wqv_ref[c], preferred_element_type=jnp.float32)
        os_ref[c] = jnp.dot(h, ws_ref[c], preferred_element_type=jnp.float32)


_BN = 400
_proj = pl.pallas_call(
    _proj_body,
    grid=(_N // _BN,),
    in_specs=[
        pl.BlockSpec((2, _BN, _H), lambda i: (0, i, 0)),
        pl.BlockSpec((_NC, _D, _H), lambda i: (0, 0, 0)),
        pl.BlockSpec((_NC, _D, 2 * _H), lambda i: (0, 0, 0)),
        pl.BlockSpec((_NC, _D, _H), lambda i: (0, 0, 0)),
    ],
    out_specs=[
        pl.BlockSpec((_NC, _BN, _H), lambda i: (0, i, 0)),
        pl.BlockSpec((_NC, _BN, 2 * _H), lambda i: (0, i, 0)),
        pl.BlockSpec((_NC, _BN, _H), lambda i: (0, i, 0)),
    ],
    out_shape=[
        jax.ShapeDtypeStruct((_NC, _N, _H), jnp.float32),
        jax.ShapeDtypeStruct((_NC, _N, 2 * _H), jnp.float32),
        jax.ShapeDtypeStruct((_NC, _N, _H), jnp.float32),
    ],
)


def _edge_body(ktab, qvtab, src_hbm, dst_hbm, skip, bias, out,
               dst_a, dst_b, ik_a, ik_b, iqv_a, iqv_b, srcv,
               kd_a, kd_b, qv_a, qv_b, bvec,
               agg, sem_a, sem_b):
    c = lax.axis_index("c")
    s = lax.axis_index("s")
    zero16 = jnp.zeros((16,), jnp.float32)
    nbase = c * _N

    def _zb(i, _):
        kd_a[i // _F16, pl.ds((i % _F16) * 16, 16)] = zero16
        return 0
    lax.fori_loop(0, _CE * _F16, _zb, 0)
    for k in range(_ZB_ITERS):
        cid = s + _NS * k
        @pl.when(cid < _NCHUNK)
        def _():
            pltpu.sync_copy(kd_a, agg.at[pl.ds(cid * _CE, _CE)])
    plsc.subcore_barrier()

    def _load(k, dst_v, ik, iqv, kd, qv, sem):
        base = s * _EPT + k * _CE
        pltpu.sync_copy(src_hbm.at[pl.ds(base, _CE)], srcv.at[0])
        pltpu.sync_copy(dst_hbm.at[pl.ds(base, _CE)], dst_v.at[0])
        for j in range(_CE // 16):
            sl = pl.ds(j * 16, 16)
            ik[0, sl] = jnp.minimum(dst_v[0, sl], _N - 1) + nbase
            iqv[0, sl] = srcv[0, sl] + nbase
        pltpu.async_copy(ktab.at[ik.at[0]], kd, sem)
        pltpu.async_copy(qvtab.at[iqv.at[0]], qv, sem)

    def _finish(dst_v, ik, iqv, kd, qv, sem):
        pltpu.make_async_copy(ktab.at[ik.at[0]], kd, sem).wait()
        pltpu.make_async_copy(qvtab.at[iqv.at[0]], qv, sem).wait()

        def _edge(e, _):
            for f in range(_F16):
                sl = pl.ds(f * 16, 16)
                z = kd[e, sl] + qv[e, sl]
                kd[e, sl] = qv[e, pl.ds(_H + f * 16, 16)] / (1.0 + jnp.exp(-z))
            return 0
        lax.fori_loop(0, _CE, _edge, 0)
        pltpu.sync_copy(kd, agg.at[dst_v.at[0]], add=True)

    bufa = (dst_a, ik_a, iqv_a, kd_a, qv_a, sem_a)
    bufb = (dst_b, ik_b, iqv_b, kd_b, qv_b, sem_b)

    def _round(r, _):
        _load(r, *bufa)
        _finish(*bufa)
        return 0

    lax.fori_loop(0, _NCHUNK, _round, 0)
    plsc.subcore_barrier()

    pltpu.sync_copy(bias.at[c], bvec)
    for k in range(_WB_ITERS):
        cid = s + _NS * k
        @pl.when(cid < _NWB)
        def _():
            r0 = cid * _WB
            pltpu.sync_copy(agg.at[pl.ds(r0, _WB)], kd_a.at[pl.ds(0, _WB)])
            pltpu.sync_copy(skip.at[c, pl.ds(r0, _WB)], kd_a.at[pl.ds(_WB, _WB)])

            def _wbloop(i, _):
                r = i // _F16
                sl = pl.ds((i % _F16) * 16, 16)
                v = kd_a[r, sl] + kd_a[_WB + r, sl] + bvec[sl]
                kd_a[r, sl] = jnp.maximum(v, 0.0)
                return 0
            lax.fori_loop(0, _WB * _F16, _wbloop, 0)
            pltpu.sync_copy(kd_a.at[pl.ds(0, _WB)], out.at[c, pl.ds(r0, _WB)])


_edge_call = functools.partial(
    pl.kernel,
    out_type=jax.ShapeDtypeStruct((_NC, _N, _H), jnp.float32),
    mesh=plsc.VectorSubcoreMesh(core_axis_name="c", subcore_axis_name="s"),
    scratch_types=[
        pltpu.VMEM((1, _CE), jnp.int32),
        pltpu.VMEM((1, _CE), jnp.int32),
        pltpu.VMEM((1, _CE), jnp.int32),
        pltpu.VMEM((1, _CE), jnp.int32),
        pltpu.VMEM((1, _CE), jnp.int32),
        pltpu.VMEM((1, _CE), jnp.int32),
        pltpu.VMEM((1, _CE), jnp.int32),
        pltpu.VMEM((_CE, _H), jnp.float32),
        pltpu.VMEM((_CE, _H), jnp.float32),
        pltpu.VMEM((_CE, 2 * _H), jnp.float32),
        pltpu.VMEM((_CE, 2 * _H), jnp.float32),
        pltpu.VMEM((_H,), jnp.float32),
        pltpu.VMEM_SHARED((_AROWS, _H), jnp.float32),
        pltpu.SemaphoreType.DMA,
        pltpu.SemaphoreType.DMA,
    ],
)(_edge_body)


def _layer(h2, src, dst, Wk, Wq, Wv, Ws, b):
    wk = Wk.reshape(_D, _NC, _H).transpose(1, 0, 2)
    ws = Ws.reshape(_D, _NC, _H).transpose(1, 0, 2)
    q3 = Wq.reshape(_D, _NC, _H)
    v3 = Wv.reshape(_D, _NC, _H)
    wqv = jnp.concatenate([q3, v3], axis=2).transpose(1, 0, 2)
    ktab, qvtab, skip = _proj(h2, wk, wqv, ws)
    return _edge_call(ktab.reshape(_NC * _N, _H), qvtab.reshape(_NC * _N, 2 * _H),
                      src, dst, skip, b.reshape(_NC, _H))


def kernel(x, edge_index, Wk0, Wq0, Wv0, Ws0, Wk1, Wq1, Wv1, Ws1, b0, b1):
    npad = _EPAD - _E
    src = jnp.concatenate([edge_index[0], jnp.zeros((npad,), jnp.int32)])
    dst = jnp.concatenate([edge_index[1], jnp.full((npad,), _N, jnp.int32)])
    h2 = x.reshape(_N, _NC, _H).transpose(1, 0, 2)
    h2 = _layer(h2, src, dst, Wk0, Wq0, Wv0, Ws0, b0)
    h2 = _layer(h2, src, dst, Wk1, Wq1, Wv1, Ws1, b1)
    return jnp.concatenate([h2[0], h2[1]], axis=1)


# speedup vs baseline: 3.9429x; 3.9429x over previous
import functools

import jax
import jax.numpy as jnp
from jax import lax
from jax.experimental import pallas as pl
from jax.experimental.pallas import tpu as pltpu
from jax.experimental.pallas import tpu_sc as plsc

_N = 10000
_E = 160000
_D = 256
_H = 128
_NC = 2
_NS = 16
_CE = 64
_NCHUNK = 157
_EPT = _NCHUNK * _CE
_EPAD = _NS * _EPT
_AROWS = _NCHUNK * _CE
_WB = 16
_NWB = _N // _WB
_WB_ITERS = (_NWB + _NS - 1) // _NS
_ZB_ITERS = (_NCHUNK + _NS - 1) // _NS
_F16 = _H // 16


def _proj_body(h_ref, wk_ref, wq_ref, wv_ref, ws_ref,
               ok_ref, oq_ref, ov_ref, os_ref):
    h = jnp.concatenate([h_ref[0], h_ref[1]], axis=1)
    for c in range(_NC):
        ok_ref[c] = jnp.dot(h, wk_ref[c], preferred_element_type=jnp.float32)
        oq_ref[c] = jnp.dot(h, wq_ref[c], preferred_element_type=jnp.float32)
        ov_ref[c] = jnp.dot(h, wv_ref[c], preferred_element_type=jnp.float32)
        os_ref[c] = jnp.dot(h, ws_ref[c], preferred_element_type=jnp.float32)


_BN = 400
_w_spec = pl.BlockSpec((_NC, _D, _H), lambda i: (0, 0, 0))
_o_spec = pl.BlockSpec((_NC, _BN, _H), lambda i: (0, i, 0))
_o_shape = jax.ShapeDtypeStruct((_NC, _N, _H), jnp.float32)
_proj = pl.pallas_call(
    _proj_body,
    grid=(_N // _BN,),
    in_specs=[pl.BlockSpec((2, _BN, _H), lambda i: (0, i, 0)),
              _w_spec, _w_spec, _w_spec, _w_spec],
    out_specs=[_o_spec, _o_spec, _o_spec, _o_spec],
    out_shape=[_o_shape, _o_shape, _o_shape, _o_shape],
)


def _edge_body(ktab, qtab, vtab, src_hbm, dst_hbm, skip, bias, out,
               dst_a, dst_b, ik_a, ik_b, iq_a, iq_b,
               kd_a, kd_b, q_a, q_b, v_a, v_b, bvec,
               agg, sem_a, sem_b):
    c = lax.axis_index("c")
    s = lax.axis_index("s")
    zero16 = jnp.zeros((16,), jnp.float32)
    nbase = c * _N

    def _zb(i, _):
        kd_a[i // _F16, pl.ds((i % _F16) * 16, 16)] = zero16
        return 0
    lax.fori_loop(0, _CE * _F16, _zb, 0)
    for k in range(_ZB_ITERS):
        cid = s + _NS * k
        @pl.when(cid < _NCHUNK)
        def _():
            pltpu.sync_copy(kd_a, agg.at[pl.ds(cid * _CE, _CE)])
    plsc.subcore_barrier()

    def _load(k, dst_v, ik, iq, kd, q, v, sem):
        base = s * _EPT + k * _CE
        pltpu.sync_copy(src_hbm.at[pl.ds(base, _CE)], iq.at[0])
        pltpu.sync_copy(dst_hbm.at[pl.ds(base, _CE)], dst_v.at[0])
        for j in range(_CE // 16):
            sl = pl.ds(j * 16, 16)
            ik[0, sl] = jnp.minimum(dst_v[0, sl], _N - 1) + nbase
            iq[0, sl] = iq[0, sl] + nbase
        pltpu.async_copy(ktab.at[ik.at[0]], kd, sem)
        pltpu.async_copy(qtab.at[iq.at[0]], q, sem)
        pltpu.async_copy(vtab.at[iq.at[0]], v, sem)

    def _finish(dst_v, ik, iq, kd, q, v, sem):
        pltpu.make_async_copy(ktab.at[ik.at[0]], kd, sem).wait()
        pltpu.make_async_copy(qtab.at[iq.at[0]], q, sem).wait()
        pltpu.make_async_copy(vtab.at[iq.at[0]], v, sem).wait()

        def _edge(e, _):
            for f in range(_F16):
                sl = pl.ds(f * 16, 16)
                z = kd[e, sl] + q[e, sl]
                kd[e, sl] = v[e, sl] / (1.0 + jnp.exp(-z))
            return 0
        lax.fori_loop(0, _CE, _edge, 0)
        pltpu.sync_copy(kd, agg.at[dst_v.at[0]], add=True)

    bufa = (dst_a, ik_a, iq_a, kd_a, q_a, v_a, sem_a)
    bufb = (dst_b, ik_b, iq_b, kd_b, q_b, v_b, sem_b)

    _load(0, *bufa)

    def _round(r, _):
        _load(2 * r + 1, *bufb)
        _finish(*bufa)
        _load(2 * r + 2, *bufa)
        _finish(*bufb)
        return 0

    lax.fori_loop(0, (_NCHUNK - 1) // 2, _round, 0)
    _finish(*bufa)
    plsc.subcore_barrier()

    pltpu.sync_copy(bias.at[c], bvec)
    for k in range(_WB_ITERS):
        cid = s + _NS * k
        @pl.when(cid < _NWB)
        def _():
            r0 = cid * _WB
            pltpu.sync_copy(agg.at[pl.ds(r0, _WB)], kd_a.at[pl.ds(0, _WB)])
            pltpu.sync_copy(skip.at[c, pl.ds(r0, _WB)], kd_a.at[pl.ds(_WB, _WB)])

            def _wbloop(i, _):
                r = i // _F16
                sl = pl.ds((i % _F16) * 16, 16)
                vv = kd_a[r, sl] + kd_a[_WB + r, sl] + bvec[sl]
                kd_a[r, sl] = jnp.maximum(vv, 0.0)
                return 0
            lax.fori_loop(0, _WB * _F16, _wbloop, 0)
            pltpu.sync_copy(kd_a.at[pl.ds(0, _WB)], out.at[c, pl.ds(r0, _WB)])


_edge_call = functools.partial(
    pl.kernel,
    out_type=jax.ShapeDtypeStruct((_NC, _N, _H), jnp.float32),
    mesh=plsc.VectorSubcoreMesh(core_axis_name="c", subcore_axis_name="s"),
    scratch_types=[
        pltpu.VMEM((1, _CE), jnp.int32),
        pltpu.VMEM((1, _CE), jnp.int32),
        pltpu.VMEM((1, _CE), jnp.int32),
        pltpu.VMEM((1, _CE), jnp.int32),
        pltpu.VMEM((1, _CE), jnp.int32),
        pltpu.VMEM((1, _CE), jnp.int32),
        pltpu.VMEM((_CE, _H), jnp.float32),
        pltpu.VMEM((_CE, _H), jnp.float32),
        pltpu.VMEM((_CE, _H), jnp.float32),
        pltpu.VMEM((_CE, _H), jnp.float32),
        pltpu.VMEM((_CE, _H), jnp.float32),
        pltpu.VMEM((_CE, _H), jnp.float32),
        pltpu.VMEM((_H,), jnp.float32),
        pltpu.VMEM_SHARED((_AROWS, _H), jnp.float32),
        pltpu.SemaphoreType.DMA,
        pltpu.SemaphoreType.DMA,
    ],
)(_edge_body)


def _layer(h2, src, dst, Wk, Wq, Wv, Ws, b):
    wk = Wk.reshape(_D, _NC, _H).transpose(1, 0, 2)
    wq = Wq.reshape(_D, _NC, _H).transpose(1, 0, 2)
    wv = Wv.reshape(_D, _NC, _H).transpose(1, 0, 2)
    ws = Ws.reshape(_D, _NC, _H).transpose(1, 0, 2)
    ktab, qtab, vtab, skip = _proj(h2, wk, wq, wv, ws)
    return _edge_call(ktab.reshape(_NC * _N, _H), qtab.reshape(_NC * _N, _H),
                      vtab.reshape(_NC * _N, _H),
                      src, dst, skip, b.reshape(_NC, _H))


def kernel(x, edge_index, Wk0, Wq0, Wv0, Ws0, Wk1, Wq1, Wv1, Ws1, b0, b1):
    npad = _EPAD - _E
    src = jnp.concatenate([edge_index[0], jnp.zeros((npad,), jnp.int32)])
    dst = jnp.concatenate([edge_index[1], jnp.full((npad,), _N, jnp.int32)])
    h2 = x.reshape(_N, _NC, _H).transpose(1, 0, 2)
    h2 = _layer(h2, src, dst, Wk0, Wq0, Wv0, Ws0, b0)
    h2 = _layer(h2, src, dst, Wk1, Wq1, Wv1, Ws1, b1)
    return jnp.concatenate([h2[0], h2[1]], axis=1)


# speedup vs baseline: 4.4682x; 1.1332x over previous
import functools

import jax
import jax.numpy as jnp
from jax import lax
from jax.experimental import pallas as pl
from jax.experimental.pallas import tpu as pltpu
from jax.experimental.pallas import tpu_sc as plsc

_N = 10000
_E = 160000
_D = 256
_H = 128
_NC = 2
_NS = 16
_CE = 64
_NCHUNK = 157
_EPT = _NCHUNK * _CE
_EPAD = _NS * _EPT
_AROWS = _NCHUNK * _CE
_WB = 16
_NWB = _N // _WB
_WB_ITERS = (_NWB + _NS - 1) // _NS
_ZB_ITERS = (_NCHUNK + _NS - 1) // _NS
_F16 = _H // 16


def _proj_body(h_ref, wk_ref, wq_ref, wv_ref, ws_ref,
               ok_ref, oq_ref, ov_ref, os_ref):
    h = jnp.concatenate([h_ref[0], h_ref[1]], axis=1)
    for c in range(_NC):
        k = jnp.dot(h, wk_ref[c], preferred_element_type=jnp.float32)
        ok_ref[c] = jnp.exp(-jnp.clip(k, -80.0, 80.0))
        q = jnp.dot(h, wq_ref[c], preferred_element_type=jnp.float32)
        oq_ref[c] = jnp.exp(-jnp.clip(q, -80.0, 80.0))
        ov_ref[c] = jnp.dot(h, wv_ref[c], preferred_element_type=jnp.float32)
        os_ref[c] = jnp.dot(h, ws_ref[c], preferred_element_type=jnp.float32)


_BN = 400
_w_spec = pl.BlockSpec((_NC, _D, _H), lambda i: (0, 0, 0))
_o_spec = pl.BlockSpec((_NC, _BN, _H), lambda i: (0, i, 0))
_o_shape = jax.ShapeDtypeStruct((_NC, _N, _H), jnp.float32)
_proj = pl.pallas_call(
    _proj_body,
    grid=(_N // _BN,),
    in_specs=[pl.BlockSpec((2, _BN, _H), lambda i: (0, i, 0)),
              _w_spec, _w_spec, _w_spec, _w_spec],
    out_specs=[_o_spec, _o_spec, _o_spec, _o_spec],
    out_shape=[_o_shape, _o_shape, _o_shape, _o_shape],
)


def _edge_body(ktab, qtab, vtab, src_hbm, dst_hbm, skip, bias, out,
               dst_a, dst_b, ik_a, ik_b, iq_a, iq_b,
               kd_a, kd_b, q_a, q_b, v_a, v_b, bvec,
               agg, sem_a, sem_b, ssem_a, ssem_b):
    c = lax.axis_index("c")
    s = lax.axis_index("s")
    zero16 = jnp.zeros((16,), jnp.float32)
    nbase = c * _N

    def _zb(i, _):
        kd_a[i // _F16, pl.ds((i % _F16) * 16, 16)] = zero16
        return 0
    lax.fori_loop(0, _CE * _F16, _zb, 0)
    for k in range(_ZB_ITERS):
        cid = s + _NS * k
        @pl.when(cid < _NCHUNK)
        def _():
            pltpu.sync_copy(kd_a, agg.at[pl.ds(cid * _CE, _CE)])
    plsc.subcore_barrier()

    def _load(k, dst_v, ik, iq, kd, q, v, sem, ssem):
        @pl.when(k >= 2)
        def _():
            pltpu.make_async_copy(kd, agg.at[dst_v.at[0]], ssem).wait()
        base = s * _EPT + k * _CE
        pltpu.sync_copy(src_hbm.at[pl.ds(base, _CE)], iq.at[0])
        pltpu.sync_copy(dst_hbm.at[pl.ds(base, _CE)], dst_v.at[0])
        for j in range(_CE // 16):
            sl = pl.ds(j * 16, 16)
            ik[0, sl] = jnp.minimum(dst_v[0, sl], _N - 1) + nbase
            iq[0, sl] = iq[0, sl] + nbase
        pltpu.async_copy(ktab.at[ik.at[0]], kd, sem)
        pltpu.async_copy(qtab.at[iq.at[0]], q, sem)
        pltpu.async_copy(vtab.at[iq.at[0]], v, sem)

    def _finish(dst_v, ik, iq, kd, q, v, sem, ssem):
        pltpu.make_async_copy(ktab.at[ik.at[0]], kd, sem).wait()
        pltpu.make_async_copy(qtab.at[iq.at[0]], q, sem).wait()
        pltpu.make_async_copy(vtab.at[iq.at[0]], v, sem).wait()

        def _edge(e, _):
            for f in range(_F16):
                sl = pl.ds(f * 16, 16)
                den = 1.0 + kd[e, sl] * q[e, sl]
                kd[e, sl] = v[e, sl] / den
            return 0
        lax.fori_loop(0, _CE, _edge, 0)
        pltpu.async_copy(kd, agg.at[dst_v.at[0]], ssem, add=True)

    bufa = (dst_a, ik_a, iq_a, kd_a, q_a, v_a, sem_a, ssem_a)
    bufb = (dst_b, ik_b, iq_b, kd_b, q_b, v_b, sem_b, ssem_b)

    _load(jnp.int32(0), *bufa)

    def _round(r, _):
        _load(2 * r + 1, *bufb)
        _finish(*bufa)
        _load(2 * r + 2, *bufa)
        _finish(*bufb)
        return 0

    lax.fori_loop(0, (_NCHUNK - 1) // 2, _round, 0)
    _finish(*bufa)
    pltpu.make_async_copy(kd_b, agg.at[dst_b.at[0]], ssem_b).wait()
    pltpu.make_async_copy(kd_a, agg.at[dst_a.at[0]], ssem_a).wait()
    plsc.subcore_barrier()

    pltpu.sync_copy(bias.at[c], bvec)
    for k in range(_WB_ITERS):
        cid = s + _NS * k
        @pl.when(cid < _NWB)
        def _():
            r0 = cid * _WB
            pltpu.sync_copy(agg.at[pl.ds(r0, _WB)], kd_a.at[pl.ds(0, _WB)])
            pltpu.sync_copy(skip.at[c, pl.ds(r0, _WB)], kd_a.at[pl.ds(_WB, _WB)])

            def _wbloop(i, _):
                r = i // _F16
                sl = pl.ds((i % _F16) * 16, 16)
                vv = kd_a[r, sl] + kd_a[_WB + r, sl] + bvec[sl]
                kd_a[r, sl] = jnp.maximum(vv, 0.0)
                return 0
            lax.fori_loop(0, _WB * _F16, _wbloop, 0)
            pltpu.sync_copy(kd_a.at[pl.ds(0, _WB)], out.at[c, pl.ds(r0, _WB)])


_edge_call = functools.partial(
    pl.kernel,
    out_type=jax.ShapeDtypeStruct((_NC, _N, _H), jnp.float32),
    mesh=plsc.VectorSubcoreMesh(core_axis_name="c", subcore_axis_name="s"),
    scratch_types=[
        pltpu.VMEM((1, _CE), jnp.int32),
        pltpu.VMEM((1, _CE), jnp.int32),
        pltpu.VMEM((1, _CE), jnp.int32),
        pltpu.VMEM((1, _CE), jnp.int32),
        pltpu.VMEM((1, _CE), jnp.int32),
        pltpu.VMEM((1, _CE), jnp.int32),
        pltpu.VMEM((_CE, _H), jnp.float32),
        pltpu.VMEM((_CE, _H), jnp.float32),
        pltpu.VMEM((_CE, _H), jnp.float32),
        pltpu.VMEM((_CE, _H), jnp.float32),
        pltpu.VMEM((_CE, _H), jnp.float32),
        pltpu.VMEM((_CE, _H), jnp.float32),
        pltpu.VMEM((_H,), jnp.float32),
        pltpu.VMEM_SHARED((_AROWS, _H), jnp.float32),
        pltpu.SemaphoreType.DMA,
        pltpu.SemaphoreType.DMA,
        pltpu.SemaphoreType.DMA,
        pltpu.SemaphoreType.DMA,
    ],
)(_edge_body)


def _layer(h2, src, dst, Wk, Wq, Wv, Ws, b):
    wk = Wk.reshape(_D, _NC, _H).transpose(1, 0, 2)
    wq = Wq.reshape(_D, _NC, _H).transpose(1, 0, 2)
    wv = Wv.reshape(_D, _NC, _H).transpose(1, 0, 2)
    ws = Ws.reshape(_D, _NC, _H).transpose(1, 0, 2)
    ktab, qtab, vtab, skip = _proj(h2, wk, wq, wv, ws)
    return _edge_call(ktab.reshape(_NC * _N, _H), qtab.reshape(_NC * _N, _H),
                      vtab.reshape(_NC * _N, _H),
                      src, dst, skip, b.reshape(_NC, _H))


def kernel(x, edge_index, Wk0, Wq0, Wv0, Ws0, Wk1, Wq1, Wv1, Ws1, b0, b1):
    npad = _EPAD - _E
    src = jnp.concatenate([edge_index[0], jnp.zeros((npad,), jnp.int32)])
    dst = jnp.concatenate([edge_index[1], jnp.full((npad,), _N, jnp.int32)])
    h2 = x.reshape(_N, _NC, _H).transpose(1, 0, 2)
    h2 = _layer(h2, src, dst, Wk0, Wq0, Wv0, Ws0, b0)
    h2 = _layer(h2, src, dst, Wk1, Wq1, Wv1, Ws1, b1)
    return jnp.concatenate([h2[0], h2[1]], axis=1)


# speedup vs baseline: 5.4076x; 1.2102x over previous
import functools

import jax
import jax.numpy as jnp
from jax import lax
from jax.experimental import pallas as pl
from jax.experimental.pallas import tpu as pltpu
from jax.experimental.pallas import tpu_sc as plsc

_N = 10000
_E = 160000
_D = 256
_H = 128
_NC = 2
_NS = 16
_CE = 64
_NCHUNK = 157
_EPT = _NCHUNK * _CE
_EPAD = _NS * _EPT
_AROWS = _NCHUNK * _CE
_WB = 16
_NWB = _N // _WB
_WB_ITERS = (_NWB + _NS - 1) // _NS
_ZB_ITERS = (_NCHUNK + _NS - 1) // _NS
_F16 = _H // 16


def _proj_body(h_ref, wk_ref, wq_ref, wv_ref, ws_ref,
               ok_ref, oq_ref, ov_ref, os_ref):
    h = jnp.concatenate([h_ref[0], h_ref[1]], axis=1)
    for c in range(_NC):
        k = jnp.dot(h, wk_ref[c], preferred_element_type=jnp.float32)
        ok_ref[c] = jnp.exp(-jnp.clip(k, -80.0, 80.0))
        q = jnp.dot(h, wq_ref[c], preferred_element_type=jnp.float32)
        oq_ref[c] = jnp.exp(-jnp.clip(q, -80.0, 80.0))
        ov_ref[c] = jnp.dot(h, wv_ref[c], preferred_element_type=jnp.float32)
        os_ref[c] = jnp.dot(h, ws_ref[c], preferred_element_type=jnp.float32)


_BN = 400
_w_spec = pl.BlockSpec((_NC, _D, _H), lambda i: (0, 0, 0))
_o_spec = pl.BlockSpec((_NC, _BN, _H), lambda i: (0, i, 0))
_o_shape = jax.ShapeDtypeStruct((_NC, _N, _H), jnp.float32)
_proj = pl.pallas_call(
    _proj_body,
    grid=(_N // _BN,),
    in_specs=[pl.BlockSpec((2, _BN, _H), lambda i: (0, i, 0)),
              _w_spec, _w_spec, _w_spec, _w_spec],
    out_specs=[_o_spec, _o_spec, _o_spec, _o_spec],
    out_shape=[_o_shape, _o_shape, _o_shape, _o_shape],
)


def _edge_body(ktab, qtab, vtab, src_hbm, dst_hbm, skip, bias, out,
               dst_a, dst_b, ik_a, ik_b, iq_a, iq_b,
               srcr_a, srcr_b, dstr_a, dstr_b,
               kd_a, kd_b, q_a, q_b, v_a, v_b, bvec,
               agg, sem_a, sem_b, ssem_a, ssem_b, isem_a, isem_b):
    c = lax.axis_index("c")
    s = lax.axis_index("s")
    zero16 = jnp.zeros((16,), jnp.float32)
    nbase = c * _N

    def _zb(i, _):
        kd_a[i // _F16, pl.ds((i % _F16) * 16, 16)] = zero16
        return 0
    lax.fori_loop(0, _CE * _F16, _zb, 0)
    for k in range(_ZB_ITERS):
        cid = s + _NS * k
        @pl.when(cid < _NCHUNK)
        def _():
            pltpu.sync_copy(kd_a, agg.at[pl.ds(cid * _CE, _CE)])
    plsc.subcore_barrier()

    def _idx_fetch(k, srcr, dstr, isem):
        base = s * _EPT + k * _CE
        pltpu.async_copy(src_hbm.at[pl.ds(base, _CE)], srcr.at[0], isem)
        pltpu.async_copy(dst_hbm.at[pl.ds(base, _CE)], dstr.at[0], isem)

    def _load(k, dst_v, ik, iq, srcr, dstr, kd, q, v, sem, ssem, isem):
        @pl.when(k >= 2)
        def _():
            pltpu.make_async_copy(kd, agg.at[dst_v.at[0]], ssem).wait()
        base = s * _EPT + k * _CE
        pltpu.make_async_copy(src_hbm.at[pl.ds(base, _CE)], srcr.at[0], isem).wait()
        pltpu.make_async_copy(dst_hbm.at[pl.ds(base, _CE)], dstr.at[0], isem).wait()
        for j in range(_CE // 16):
            sl = pl.ds(j * 16, 16)
            dv = dstr[0, sl]
            dst_v[0, sl] = dv
            ik[0, sl] = jnp.minimum(dv, _N - 1) + nbase
            iq[0, sl] = srcr[0, sl] + nbase
        pltpu.async_copy(ktab.at[ik.at[0]], kd, sem)
        pltpu.async_copy(qtab.at[iq.at[0]], q, sem)
        pltpu.async_copy(vtab.at[iq.at[0]], v, sem)
        @pl.when(k + 2 < _NCHUNK)
        def _():
            _idx_fetch(k + 2, srcr, dstr, isem)

    def _finish(dst_v, ik, iq, srcr, dstr, kd, q, v, sem, ssem, isem):
        pltpu.make_async_copy(ktab.at[ik.at[0]], kd, sem).wait()
        pltpu.make_async_copy(qtab.at[iq.at[0]], q, sem).wait()
        pltpu.make_async_copy(vtab.at[iq.at[0]], v, sem).wait()

        def _edge(e, _):
            for f in range(_F16):
                sl = pl.ds(f * 16, 16)
                den = 1.0 + kd[e, sl] * q[e, sl]
                kd[e, sl] = v[e, sl] / den
            return 0
        lax.fori_loop(0, _CE, _edge, 0)
        pltpu.async_copy(kd, agg.at[dst_v.at[0]], ssem, add=True)

    bufa = (dst_a, ik_a, iq_a, srcr_a, dstr_a, kd_a, q_a, v_a,
            sem_a, ssem_a, isem_a)
    bufb = (dst_b, ik_b, iq_b, srcr_b, dstr_b, kd_b, q_b, v_b,
            sem_b, ssem_b, isem_b)

    _idx_fetch(0, srcr_a, dstr_a, isem_a)
    _idx_fetch(1, srcr_b, dstr_b, isem_b)
    _load(jnp.int32(0), *bufa)

    def _round(r, _):
        _load(2 * r + 1, *bufb)
        _finish(*bufa)
        _load(2 * r + 2, *bufa)
        _finish(*bufb)
        return 0

    lax.fori_loop(0, (_NCHUNK - 1) // 2, _round, 0)
    _finish(*bufa)
    pltpu.make_async_copy(kd_b, agg.at[dst_b.at[0]], ssem_b).wait()
    pltpu.make_async_copy(kd_a, agg.at[dst_a.at[0]], ssem_a).wait()
    plsc.subcore_barrier()

    pltpu.sync_copy(bias.at[c], bvec)
    for k in range(_WB_ITERS):
        cid = s + _NS * k
        @pl.when(cid < _NWB)
        def _():
            r0 = cid * _WB
            pltpu.sync_copy(agg.at[pl.ds(r0, _WB)], kd_a.at[pl.ds(0, _WB)])
            pltpu.sync_copy(skip.at[c, pl.ds(r0, _WB)], kd_a.at[pl.ds(_WB, _WB)])

            def _wbloop(i, _):
                r = i // _F16
                sl = pl.ds((i % _F16) * 16, 16)
                vv = kd_a[r, sl] + kd_a[_WB + r, sl] + bvec[sl]
                kd_a[r, sl] = jnp.maximum(vv, 0.0)
                return 0
            lax.fori_loop(0, _WB * _F16, _wbloop, 0)
            pltpu.sync_copy(kd_a.at[pl.ds(0, _WB)], out.at[c, pl.ds(r0, _WB)])


_edge_call = functools.partial(
    pl.kernel,
    out_type=jax.ShapeDtypeStruct((_NC, _N, _H), jnp.float32),
    mesh=plsc.VectorSubcoreMesh(core_axis_name="c", subcore_axis_name="s"),
    scratch_types=[
        pltpu.VMEM((1, _CE), jnp.int32),
        pltpu.VMEM((1, _CE), jnp.int32),
        pltpu.VMEM((1, _CE), jnp.int32),
        pltpu.VMEM((1, _CE), jnp.int32),
        pltpu.VMEM((1, _CE), jnp.int32),
        pltpu.VMEM((1, _CE), jnp.int32),
        pltpu.VMEM((1, _CE), jnp.int32),
        pltpu.VMEM((1, _CE), jnp.int32),
        pltpu.VMEM((1, _CE), jnp.int32),
        pltpu.VMEM((1, _CE), jnp.int32),
        pltpu.VMEM((_CE, _H), jnp.float32),
        pltpu.VMEM((_CE, _H), jnp.float32),
        pltpu.VMEM((_CE, _H), jnp.float32),
        pltpu.VMEM((_CE, _H), jnp.float32),
        pltpu.VMEM((_CE, _H), jnp.float32),
        pltpu.VMEM((_CE, _H), jnp.float32),
        pltpu.VMEM((_H,), jnp.float32),
        pltpu.VMEM_SHARED((_AROWS, _H), jnp.float32),
        pltpu.SemaphoreType.DMA,
        pltpu.SemaphoreType.DMA,
        pltpu.SemaphoreType.DMA,
        pltpu.SemaphoreType.DMA,
        pltpu.SemaphoreType.DMA,
        pltpu.SemaphoreType.DMA,
    ],
)(_edge_body)


def _layer(h2, src, dst, Wk, Wq, Wv, Ws, b):
    wk = Wk.reshape(_D, _NC, _H).transpose(1, 0, 2)
    wq = Wq.reshape(_D, _NC, _H).transpose(1, 0, 2)
    wv = Wv.reshape(_D, _NC, _H).transpose(1, 0, 2)
    ws = Ws.reshape(_D, _NC, _H).transpose(1, 0, 2)
    ktab, qtab, vtab, skip = _proj(h2, wk, wq, wv, ws)
    return _edge_call(ktab.reshape(_NC * _N, _H), qtab.reshape(_NC * _N, _H),
                      vtab.reshape(_NC * _N, _H),
                      src, dst, skip, b.reshape(_NC, _H))


def kernel(x, edge_index, Wk0, Wq0, Wv0, Ws0, Wk1, Wq1, Wv1, Ws1, b0, b1):
    npad = _EPAD - _E
    src = jnp.concatenate([edge_index[0], jnp.zeros((npad,), jnp.int32)])
    dst = jnp.concatenate([edge_index[1], jnp.full((npad,), _N, jnp.int32)])
    h2 = x.reshape(_N, _NC, _H).transpose(1, 0, 2)
    h2 = _layer(h2, src, dst, Wk0, Wq0, Wv0, Ws0, b0)
    h2 = _layer(h2, src, dst, Wk1, Wq1, Wv1, Ws1, b1)
    return jnp.concatenate([h2[0], h2[1]], axis=1)


# speedup vs baseline: 5.5971x; 1.0350x over previous
import functools

import jax
import jax.numpy as jnp
from jax import lax
from jax.experimental import pallas as pl
from jax.experimental.pallas import tpu as pltpu
from jax.experimental.pallas import tpu_sc as plsc

_N = 10000
_E = 160000
_D = 256
_H = 128
_NC = 2
_NS = 16
_CE = 64
_NCHUNK = 157
_EPT = _NCHUNK * _CE
_EPAD = _NS * _EPT
_AROWS = _NCHUNK * _CE
_WB = 16
_NWB = _N // _WB
_WB_ITERS = (_NWB + _NS - 1) // _NS
_ZB_ITERS = (_NCHUNK + _NS - 1) // _NS
_F16 = _H // 16


def _proj_body(h_ref, wk_ref, wq_ref, wv_ref, ws_ref,
               ok_ref, oq_ref, ov_ref, os_ref):
    h = h_ref[...]
    for c in range(_NC):
        k = jnp.dot(h, wk_ref[c], preferred_element_type=jnp.float32)
        ok_ref[c] = jnp.exp(-jnp.clip(k, -80.0, 80.0))
        q = jnp.dot(h, wq_ref[c], preferred_element_type=jnp.float32)
        oq_ref[c] = jnp.exp(-jnp.clip(q, -80.0, 80.0))
        ov_ref[c] = jnp.dot(h, wv_ref[c], preferred_element_type=jnp.float32)
        os_ref[c] = jnp.dot(h, ws_ref[c], preferred_element_type=jnp.float32)


_BN = 400
_w_spec = pl.BlockSpec((_NC, _D, _H), lambda i: (0, 0, 0))
_o_spec = pl.BlockSpec((_NC, _BN, _H), lambda i: (0, i, 0))
_o_shape = jax.ShapeDtypeStruct((_NC, _N, _H), jnp.float32)
_proj = pl.pallas_call(
    _proj_body,
    grid=(_N // _BN,),
    in_specs=[pl.BlockSpec((_BN, _D), lambda i: (i, 0)),
              _w_spec, _w_spec, _w_spec, _w_spec],
    out_specs=[_o_spec, _o_spec, _o_spec, _o_spec],
    out_shape=[_o_shape, _o_shape, _o_shape, _o_shape],
)


def _edge_body(ktab, qtab, vtab, src_hbm, dst_hbm, skip, bias, out,
               dst_a, dst_b, ik_a, ik_b, iq_a, iq_b,
               srcr_a, srcr_b, dstr_a, dstr_b,
               kd_a, kd_b, q_a, q_b, v_a, v_b, bvec,
               agg, sem_a, sem_b, ssem_a, ssem_b, isem_a, isem_b):
    c = lax.axis_index("c")
    s = lax.axis_index("s")
    zero16 = jnp.zeros((16,), jnp.float32)
    nbase = c * _N

    def _zb(i, _):
        kd_a[i // _F16, pl.ds((i % _F16) * 16, 16)] = zero16
        return 0
    lax.fori_loop(0, _CE * _F16, _zb, 0)
    for k in range(_ZB_ITERS):
        cid = s + _NS * k
        @pl.when(cid < _NCHUNK)
        def _():
            pltpu.sync_copy(kd_a, agg.at[pl.ds(cid * _CE, _CE)])
    plsc.subcore_barrier()

    def _idx_fetch(k, srcr, dstr, isem):
        base = s * _EPT + k * _CE
        pltpu.async_copy(src_hbm.at[pl.ds(base, _CE)], srcr.at[0], isem)
        pltpu.async_copy(dst_hbm.at[pl.ds(base, _CE)], dstr.at[0], isem)

    def _load(k, dst_v, ik, iq, srcr, dstr, kd, q, v, sem, ssem, isem):
        @pl.when(k >= 2)
        def _():
            pltpu.make_async_copy(kd, agg.at[dst_v.at[0]], ssem).wait()
        base = s * _EPT + k * _CE
        pltpu.make_async_copy(src_hbm.at[pl.ds(base, _CE)], srcr.at[0], isem).wait()
        pltpu.make_async_copy(dst_hbm.at[pl.ds(base, _CE)], dstr.at[0], isem).wait()
        for j in range(_CE // 16):
            sl = pl.ds(j * 16, 16)
            dv = dstr[0, sl]
            dst_v[0, sl] = dv
            ik[0, sl] = jnp.minimum(dv, _N - 1) + nbase
            iq[0, sl] = srcr[0, sl] + nbase
        pltpu.async_copy(ktab.at[ik.at[0]], kd, sem)
        pltpu.async_copy(qtab.at[iq.at[0]], q, sem)
        pltpu.async_copy(vtab.at[iq.at[0]], v, sem)
        @pl.when(k + 2 < _NCHUNK)
        def _():
            _idx_fetch(k + 2, srcr, dstr, isem)

    def _finish(dst_v, ik, iq, srcr, dstr, kd, q, v, sem, ssem, isem):
        pltpu.make_async_copy(ktab.at[ik.at[0]], kd, sem).wait()
        pltpu.make_async_copy(qtab.at[iq.at[0]], q, sem).wait()
        pltpu.make_async_copy(vtab.at[iq.at[0]], v, sem).wait()

        def _edge(e, _):
            for f in range(_F16):
                sl = pl.ds(f * 16, 16)
                den = 1.0 + kd[e, sl] * q[e, sl]
                kd[e, sl] = v[e, sl] / den
            return 0
        lax.fori_loop(0, _CE, _edge, 0)
        pltpu.async_copy(kd, agg.at[dst_v.at[0]], ssem, add=True)

    bufa = (dst_a, ik_a, iq_a, srcr_a, dstr_a, kd_a, q_a, v_a,
            sem_a, ssem_a, isem_a)
    bufb = (dst_b, ik_b, iq_b, srcr_b, dstr_b, kd_b, q_b, v_b,
            sem_b, ssem_b, isem_b)

    _idx_fetch(0, srcr_a, dstr_a, isem_a)
    _idx_fetch(1, srcr_b, dstr_b, isem_b)
    _load(jnp.int32(0), *bufa)

    def _round(r, _):
        _load(2 * r + 1, *bufb)
        _finish(*bufa)
        _load(2 * r + 2, *bufa)
        _finish(*bufb)
        return 0

    lax.fori_loop(0, (_NCHUNK - 1) // 2, _round, 0)
    _finish(*bufa)
    pltpu.make_async_copy(kd_b, agg.at[dst_b.at[0]], ssem_b).wait()
    pltpu.make_async_copy(kd_a, agg.at[dst_a.at[0]], ssem_a).wait()
    plsc.subcore_barrier()

    pltpu.sync_copy(bias.at[c], bvec)
    for k in range(_WB_ITERS):
        cid = s + _NS * k
        @pl.when(cid < _NWB)
        def _():
            r0 = cid * _WB
            pltpu.sync_copy(agg.at[pl.ds(r0, _WB)], kd_a.at[pl.ds(0, _WB)])
            pltpu.sync_copy(skip.at[c, pl.ds(r0, _WB)], kd_a.at[pl.ds(_WB, _WB)])

            def _wbloop(i, _):
                r = i // _F16
                sl = pl.ds((i % _F16) * 16, 16)
                vv = kd_a[r, sl] + kd_a[_WB + r, sl] + bvec[sl]
                kd_a[r, sl] = jnp.maximum(vv, 0.0)
                return 0
            lax.fori_loop(0, _WB * _F16, _wbloop, 0)
            pltpu.sync_copy(kd_a.at[pl.ds(0, _WB)],
                            out.at[pl.ds(r0, _WB), pl.ds(c * _H, _H)])


_edge_call = functools.partial(
    pl.kernel,
    out_type=jax.ShapeDtypeStruct((_N, _D), jnp.float32),
    mesh=plsc.VectorSubcoreMesh(core_axis_name="c", subcore_axis_name="s"),
    scratch_types=[
        pltpu.VMEM((1, _CE), jnp.int32),
        pltpu.VMEM((1, _CE), jnp.int32),
        pltpu.VMEM((1, _CE), jnp.int32),
        pltpu.VMEM((1, _CE), jnp.int32),
        pltpu.VMEM((1, _CE), jnp.int32),
        pltpu.VMEM((1, _CE), jnp.int32),
        pltpu.VMEM((1, _CE), jnp.int32),
        pltpu.VMEM((1, _CE), jnp.int32),
        pltpu.VMEM((1, _CE), jnp.int32),
        pltpu.VMEM((1, _CE), jnp.int32),
        pltpu.VMEM((_CE, _H), jnp.float32),
        pltpu.VMEM((_CE, _H), jnp.float32),
        pltpu.VMEM((_CE, _H), jnp.float32),
        pltpu.VMEM((_CE, _H), jnp.float32),
        pltpu.VMEM((_CE, _H), jnp.float32),
        pltpu.VMEM((_CE, _H), jnp.float32),
        pltpu.VMEM((_H,), jnp.float32),
        pltpu.VMEM_SHARED((_AROWS, _H), jnp.float32),
        pltpu.SemaphoreType.DMA,
        pltpu.SemaphoreType.DMA,
        pltpu.SemaphoreType.DMA,
        pltpu.SemaphoreType.DMA,
        pltpu.SemaphoreType.DMA,
        pltpu.SemaphoreType.DMA,
    ],
)(_edge_body)


def _layer(h, src, dst, Wk, Wq, Wv, Ws, b):
    wk = Wk.reshape(_D, _NC, _H).transpose(1, 0, 2)
    wq = Wq.reshape(_D, _NC, _H).transpose(1, 0, 2)
    wv = Wv.reshape(_D, _NC, _H).transpose(1, 0, 2)
    ws = Ws.reshape(_D, _NC, _H).transpose(1, 0, 2)
    ktab, qtab, vtab, skip = _proj(h, wk, wq, wv, ws)
    return _edge_call(ktab.reshape(_NC * _N, _H), qtab.reshape(_NC * _N, _H),
                      vtab.reshape(_NC * _N, _H),
                      src, dst, skip, b.reshape(_NC, _H))


def kernel(x, edge_index, Wk0, Wq0, Wv0, Ws0, Wk1, Wq1, Wv1, Ws1, b0, b1):
    npad = _EPAD - _E
    src = jnp.concatenate([edge_index[0], jnp.zeros((npad,), jnp.int32)])
    dst = jnp.concatenate([edge_index[1], jnp.full((npad,), _N, jnp.int32)])
    h = _layer(x, src, dst, Wk0, Wq0, Wv0, Ws0, b0)
    return _layer(h, src, dst, Wk1, Wq1, Wv1, Ws1, b1)
